# baseline (speedup 1.0000x reference)
import functools

import jax
import jax.numpy as jnp
from jax import lax
from jax.experimental import pallas as pl
from jax.experimental.pallas import tpu as pltpu
from jax.experimental.pallas import tpu_sc as plsc

_EMB_DIM = 64
_HALF = 32
_BATCH = 16384
_NC = 2
_NS = 16
_NW = _NC * _NS
_B_PER_W = _BATCH // _NW


def _gather_body(e0_hbm, e1_hbm, rel_hbm, h_idx_hbm, t_idx_hbm, r_idx_hbm,
                 h0_out, h1_out, t0_out, t1_out, r_out,
                 iv_h, iv_t, iv_r, h0_v, h1_v, t0_v, t1_v, r_v, sem):
    wid = lax.axis_index("s") * _NC + lax.axis_index("c")
    base = wid * _B_PER_W
    sl = pl.ds(base, _B_PER_W)
    pltpu.sync_copy(h_idx_hbm.at[sl], iv_h)
    pltpu.sync_copy(t_idx_hbm.at[sl], iv_t)
    pltpu.sync_copy(r_idx_hbm.at[sl], iv_r)
    copies = [
        pltpu.async_copy(e0_hbm.at[iv_h], h0_v, sem),
        pltpu.async_copy(e1_hbm.at[iv_h], h1_v, sem),
        pltpu.async_copy(e0_hbm.at[iv_t], t0_v, sem),
        pltpu.async_copy(e1_hbm.at[iv_t], t1_v, sem),
        pltpu.async_copy(rel_hbm.at[iv_r], r_v, sem),
    ]
    for c in copies:
        c.wait()
    pltpu.sync_copy(h0_v, h0_out.at[sl])
    pltpu.sync_copy(h1_v, h1_out.at[sl])
    pltpu.sync_copy(t0_v, t0_out.at[sl])
    pltpu.sync_copy(t1_v, t1_out.at[sl])
    pltpu.sync_copy(r_v, r_out.at[sl])


_gather5 = functools.partial(
    pl.kernel,
    out_type=[jax.ShapeDtypeStruct((_BATCH, _HALF), jnp.float32)] * 4
    + [jax.ShapeDtypeStruct((_BATCH, _EMB_DIM), jnp.float32)],
    mesh=plsc.VectorSubcoreMesh(core_axis_name="c", subcore_axis_name="s"),
    scratch_types=[
        pltpu.VMEM((_B_PER_W,), jnp.int32),
        pltpu.VMEM((_B_PER_W,), jnp.int32),
        pltpu.VMEM((_B_PER_W,), jnp.int32),
        pltpu.VMEM((_B_PER_W, _HALF), jnp.float32),
        pltpu.VMEM((_B_PER_W, _HALF), jnp.float32),
        pltpu.VMEM((_B_PER_W, _HALF), jnp.float32),
        pltpu.VMEM((_B_PER_W, _HALF), jnp.float32),
        pltpu.VMEM((_B_PER_W, _EMB_DIM), jnp.float32),
        pltpu.SemaphoreType.DMA,
    ],
    compiler_params=pltpu.CompilerParams(use_tc_tiling_on_sc=False),
)(_gather_body)


_CHUNK = 2048
_NCHUNK = _BATCH // _CHUNK


def _score_body(h0_ref, h1_ref, t0_ref, t1_ref, r_ref, o_ref):
    def ss_step(i, acc):
        s = pl.ds(i * _CHUNK, _CHUNK)
        h0 = h0_ref[s, :]
        h1 = h1_ref[s, :]
        t0 = t0_ref[s, :]
        t1 = t1_ref[s, :]
        rr = r_ref[s, :]
        return (acc[0] + jnp.sum(h0 * h0), acc[1] + jnp.sum(h1 * h1),
                acc[2] + jnp.sum(t0 * t0), acc[3] + jnp.sum(t1 * t1),
                acc[4] + jnp.sum(rr[:, :_HALF] * rr[:, :_HALF]),
                acc[5] + jnp.sum(rr[:, _HALF:] * rr[:, _HALF:]))

    z = jnp.float32(0)
    sh0, sh1, st0, st1, sr0, sr1 = lax.fori_loop(
        0, _NCHUNK, ss_step, (z, z, z, z, z, z))
    d0 = jnp.sqrt(sh0) * jnp.sqrt(sr0) * jnp.sqrt(st0)
    d1 = jnp.sqrt(sh1) * jnp.sqrt(sr1) * jnp.sqrt(st1)

    def sc_step(i, carry):
        s = pl.ds(i * _CHUNK, _CHUNK)
        acc = None
        for h_ref, t_ref, half, d in ((h0_ref, t0_ref, 0, d0),
                                      (h1_ref, t1_ref, 1, d1)):
            m = h_ref[s, :] * r_ref[s, half * _HALF:(half + 1) * _HALF] * t_ref[s, :]
            p = 1.0 - m / d
            w = _HALF
            while w > 1:
                w //= 2
                p = p[:, :w] * p[:, w:2 * w]
            score = 1.0 - p[:, 0]
            acc = -score if acc is None else acc - score
        o_ref[s] = acc
        return carry

    lax.fori_loop(0, _NCHUNK, sc_step, 0)


def kernel(ent_embeddings, rel_embeddings, predict_h, predict_t, predict_r):
    h_idx = predict_h.astype(jnp.int32)
    t_idx = predict_t.astype(jnp.int32)
    r_idx = predict_r.astype(jnp.int32)
    e0 = ent_embeddings[:, :_HALF]
    e1 = ent_embeddings[:, _HALF:]
    h0, h1, t0, t1, r = _gather5(e0, e1, rel_embeddings,
                                 h_idx, t_idx, r_idx)
    pred = pl.pallas_call(
        _score_body,
        out_shape=jax.ShapeDtypeStruct((_BATCH,), jnp.float32),
    )(h0, h1, t0, t1, r)
    return pred


# speedup vs baseline: 2.1580x; 2.1580x over previous
import functools

import jax
import jax.numpy as jnp
from jax import lax
from jax.experimental import pallas as pl
from jax.experimental.pallas import tpu as pltpu
from jax.experimental.pallas import tpu_sc as plsc

_EMB_DIM = 64
_HALF = 32
_BATCH = 16384
_NC = 2
_NS = 16
_NW = _NC * _NS
_B_PER_W = _BATCH // _NW


def _gather_body(ent_hbm, rel_hbm, h_idx_hbm, t_idx_hbm, r_idx_hbm,
                 h_out, t_out, r_out,
                 h_iv, t_iv, r_iv, h_rows, t_rows, r_rows,
                 sem_h, sem_t, sem_r):
    wid = lax.axis_index("s") * _NC + lax.axis_index("c")
    base = wid * _B_PER_W
    pltpu.sync_copy(h_idx_hbm.at[pl.ds(base, _B_PER_W)], h_iv)
    pltpu.sync_copy(t_idx_hbm.at[pl.ds(base, _B_PER_W)], t_iv)
    pltpu.sync_copy(r_idx_hbm.at[pl.ds(base, _B_PER_W)], r_iv)
    ch = pltpu.async_copy(ent_hbm.at[h_iv], h_rows, sem_h)
    ct = pltpu.async_copy(ent_hbm.at[t_iv], t_rows, sem_t)
    cr = pltpu.async_copy(rel_hbm.at[r_iv], r_rows, sem_r)
    ch.wait()
    pltpu.sync_copy(h_rows, h_out.at[pl.ds(base, _B_PER_W)])
    ct.wait()
    pltpu.sync_copy(t_rows, t_out.at[pl.ds(base, _B_PER_W)])
    cr.wait()
    pltpu.sync_copy(r_rows, r_out.at[pl.ds(base, _B_PER_W)])


_gather3 = functools.partial(
    pl.kernel,
    out_type=[jax.ShapeDtypeStruct((_BATCH, _EMB_DIM), jnp.float32)] * 3,
    mesh=plsc.VectorSubcoreMesh(core_axis_name="c", subcore_axis_name="s"),
    scratch_types=[
        pltpu.VMEM((_B_PER_W,), jnp.int32),
        pltpu.VMEM((_B_PER_W,), jnp.int32),
        pltpu.VMEM((_B_PER_W,), jnp.int32),
        pltpu.VMEM((_B_PER_W, _EMB_DIM), jnp.float32),
        pltpu.VMEM((_B_PER_W, _EMB_DIM), jnp.float32),
        pltpu.VMEM((_B_PER_W, _EMB_DIM), jnp.float32),
        pltpu.SemaphoreType.DMA,
        pltpu.SemaphoreType.DMA,
        pltpu.SemaphoreType.DMA,
    ],
    compiler_params=pltpu.CompilerParams(use_tc_tiling_on_sc=False),
)(_gather_body)


_CHUNK = 2048
_NCHUNK = _BATCH // _CHUNK


def _score_body(h_ref, t_ref, r_ref, o_ref):
    def ss_step(i, acc):
        s = pl.ds(i * _CHUNK, _CHUNK)
        hh = h_ref[s, :]
        tt = t_ref[s, :]
        rr = r_ref[s, :]
        hh = hh * hh
        tt = tt * tt
        rr = rr * rr
        return (acc[0] + jnp.sum(hh[:, :_HALF]), acc[1] + jnp.sum(hh[:, _HALF:]),
                acc[2] + jnp.sum(tt[:, :_HALF]), acc[3] + jnp.sum(tt[:, _HALF:]),
                acc[4] + jnp.sum(rr[:, :_HALF]), acc[5] + jnp.sum(rr[:, _HALF:]))

    z = jnp.float32(0)
    sh0, sh1, st0, st1, sr0, sr1 = lax.fori_loop(
        0, _NCHUNK, ss_step, (z, z, z, z, z, z))
    d0 = jnp.sqrt(sh0) * jnp.sqrt(sr0) * jnp.sqrt(st0)
    d1 = jnp.sqrt(sh1) * jnp.sqrt(sr1) * jnp.sqrt(st1)

    def sc_step(i, carry):
        s = pl.ds(i * _CHUNK, _CHUNK)
        m = h_ref[s, :] * r_ref[s, :] * t_ref[s, :]
        acc = None
        for half, d in ((0, d0), (1, d1)):
            p = 1.0 - m[:, half * _HALF:(half + 1) * _HALF] / d
            w = _HALF
            while w > 1:
                w //= 2
                p = p[:, :w] * p[:, w:2 * w]
            score = 1.0 - p[:, 0]
            acc = -score if acc is None else acc - score
        o_ref[s] = acc
        return carry

    lax.fori_loop(0, _NCHUNK, sc_step, 0)


def kernel(ent_embeddings, rel_embeddings, predict_h, predict_t, predict_r):
    h_idx = predict_h.astype(jnp.int32)
    t_idx = predict_t.astype(jnp.int32)
    r_idx = predict_r.astype(jnp.int32)
    h_rows, t_rows, r_rows = _gather3(
        ent_embeddings, rel_embeddings, h_idx, t_idx, r_idx)
    pred = pl.pallas_call(
        _score_body,
        out_shape=jax.ShapeDtypeStruct((_BATCH,), jnp.float32),
    )(h_rows, t_rows, r_rows)
    return pred
